# baseline (speedup 1.0000x reference)
import functools
import math

import jax
import jax.numpy as jnp
from jax import lax
from jax.experimental import pallas as pl
from jax.experimental.pallas import tpu as pltpu
from jax.experimental.pallas import tpu_sc as plsc

SEQ = 50
BATCH = 16384
EMB = 64
VOCAB = 1000000
SCALE = math.sqrt(EMB)

NC = 2
NS = 16
NW = NC * NS

B_PER_W = BATCH // NW
CHUNK = 128
NCHUNK = B_PER_W // CHUNK

TBLK = 8192
TGRID = -(-VOCAB // TBLK)


def _relayout_body(tt_ref, eye_ref, l_ref):
    x = tt_ref[...]
    y = lax.dot_general(x, eye_ref[...],
                        (((0,), (0,)), ((), ())),
                        preferred_element_type=jnp.float32)
    l_ref[:, 0:EMB] = y


_relayout = pl.pallas_call(
    _relayout_body,
    grid=(TGRID,),
    in_specs=[
        pl.BlockSpec((EMB, TBLK), lambda i: (0, i)),
        pl.BlockSpec((EMB, EMB), lambda i: (0, 0)),
    ],
    out_specs=pl.BlockSpec((TBLK, 128), lambda i: (i, 0)),
    out_shape=jax.ShapeDtypeStruct((VOCAB, 128), jnp.float32),
)

_mesh = plsc.VectorSubcoreMesh(core_axis_name="c", subcore_axis_name="s")


@functools.partial(
    pl.kernel,
    out_type=jax.ShapeDtypeStruct((SEQ, BATCH, 128), jnp.float32),
    mesh=_mesh,
    compiler_params=pltpu.CompilerParams(use_tc_tiling_on_sc=False),
    scratch_types=(
        [pltpu.VMEM((SEQ, B_PER_W), jnp.int32)]
        + [pltpu.VMEM((CHUNK, 128), jnp.float32) for _ in range(NCHUNK)]
        + [pltpu.VMEM((SEQ, EMB), jnp.float32)]
        + [pltpu.SemaphoreType.DMA]
        + [pltpu.SemaphoreType.DMA for _ in range(NCHUNK)]
        + [pltpu.SemaphoreType.DMA for _ in range(NCHUNK)]
    ),
)
def _emb_kernel(data_hbm, table_hbm, pos_hbm, out_hbm,
                idx_all, r0, r1, r2, r3, pos_v,
                idx_sem, gs0, gs1, gs2, gs3, os0, os1, os2, os3):
    rbuf = [r0, r1, r2, r3]
    gsem = [gs0, gs1, gs2, gs3]
    osem = [os0, os1, os2, os3]

    wid = lax.axis_index("s") * NC + lax.axis_index("c")
    col0 = wid * B_PER_W

    idx_cp = pltpu.make_async_copy(
        data_hbm.at[:, pl.ds(col0, B_PER_W)], idx_all, idx_sem)
    idx_cp.start()
    pltpu.sync_copy(pos_hbm, pos_v)
    idx_cp.wait()

    def prefill(b, o):
        pv = [pos_v[o, pl.ds(16 * j, 16)] for j in range(4)]

        def row_body(r, carry):
            for j in range(4):
                rbuf[b][r, pl.ds(16 * j, 16)] = pv[j]
            return carry

        lax.fori_loop(0, CHUNK, row_body, 0, unroll=4)

    def fire_gather(b, o):
        pltpu.async_copy(
            table_hbm.at[idx_all.at[o, pl.ds(CHUNK * b, CHUNK)]],
            rbuf[b], gsem[b], add=True)

    for b in range(NCHUNK):
        prefill(b, 0)
        fire_gather(b, 0)

    def outer(o, carry):
        for b in range(NCHUNK):
            pltpu.make_async_copy(
                table_hbm.at[idx_all.at[o, pl.ds(CHUNK * b, CHUNK)]],
                rbuf[b], gsem[b]).wait()

            out_cp = pltpu.make_async_copy(
                rbuf[b],
                out_hbm.at[o, pl.ds(col0 + CHUNK * b, CHUNK), :],
                osem[b])
            out_cp.start()

            @pl.when(o < SEQ - 1)
            def _(o=o, b=b, out_cp=out_cp):
                out_cp.wait()
                prefill(b, o + 1)
                fire_gather(b, o + 1)
        return carry

    lax.fori_loop(0, SEQ, outer, 0)

    for b in range(NCHUNK):
        pltpu.make_async_copy(
            rbuf[b], out_hbm.at[0, pl.ds(0, CHUNK), :], osem[b]).wait()


def _to_tiles_body(i_ref, o_ref):
    y = i_ref[0, :, 0:EMB]
    z = jnp.swapaxes(y, 0, 1)
    z = z.reshape(8, 8, 16, 128)
    o_ref[0] = z.transpose(0, 2, 1, 3)


_to_tiles = pl.pallas_call(
    _to_tiles_body,
    grid=(SEQ, 8),
    in_specs=[pl.BlockSpec((1, 2048, 128), lambda s, j: (s, j, 0))],
    out_specs=pl.BlockSpec((1, 8, 16, 8, 128), lambda s, j: (s, 0, j, 0, 0)),
    out_shape=jax.ShapeDtypeStruct((SEQ, 8, BATCH // 128, 8, 128),
                                   jnp.float32),
)


def kernel(data, token_table, position_table):
    eye = jnp.eye(EMB, dtype=jnp.float32) * SCALE
    scaled = _relayout(token_table.T, eye)
    ipad = _emb_kernel(data.astype(jnp.int32), scaled, position_table)
    out5d = _to_tiles(ipad)
    return out5d.transpose(0, 2, 4, 1, 3).reshape(SEQ, BATCH, EMB)


# speedup vs baseline: 1.2057x; 1.2057x over previous
import functools
import math

import jax
import jax.numpy as jnp
from jax import lax
from jax.experimental import pallas as pl
from jax.experimental.pallas import tpu as pltpu
from jax.experimental.pallas import tpu_sc as plsc

SEQ = 50
BATCH = 16384
EMB = 64
VOCAB = 1000000
SCALE = math.sqrt(EMB)

NC = 2
NS = 16
NW = NC * NS

B_PER_W = BATCH // NW
CHUNK = 128
NCHUNK = B_PER_W // CHUNK

TBLK = 8192
TGRID = -(-VOCAB // TBLK)


def _relayout_body(tt_ref, eye_ref, l_ref):
    x = tt_ref[...]
    y = lax.dot_general(x, eye_ref[...],
                        (((0,), (0,)), ((), ())),
                        preferred_element_type=jnp.float32)
    l_ref[:, 0:EMB] = y


_relayout = pl.pallas_call(
    _relayout_body,
    grid=(TGRID,),
    in_specs=[
        pl.BlockSpec((EMB, TBLK), lambda i: (0, i)),
        pl.BlockSpec((EMB, EMB), lambda i: (0, 0)),
    ],
    out_specs=pl.BlockSpec((TBLK, 128), lambda i: (i, 0)),
    out_shape=jax.ShapeDtypeStruct((VOCAB, 128), jnp.float32),
)

_mesh = plsc.VectorSubcoreMesh(core_axis_name="c", subcore_axis_name="s")


@functools.partial(
    pl.kernel,
    out_type=jax.ShapeDtypeStruct((SEQ, BATCH, 128), jnp.float32),
    mesh=_mesh,
    compiler_params=pltpu.CompilerParams(use_tc_tiling_on_sc=False),
    scratch_types=(
        [pltpu.VMEM((SEQ, B_PER_W), jnp.int32)]
        + [pltpu.VMEM((CHUNK, 128), jnp.float32) for _ in range(NCHUNK)]
        + [pltpu.VMEM((SEQ, EMB), jnp.float32)]
        + [pltpu.SemaphoreType.DMA]
        + [pltpu.SemaphoreType.DMA for _ in range(NCHUNK)]
        + [pltpu.SemaphoreType.DMA for _ in range(NCHUNK)]
    ),
)
def _emb_kernel(data_hbm, table_hbm, pos_hbm, out_hbm,
                idx_all, r0, r1, r2, r3, pos_v,
                idx_sem, gs0, gs1, gs2, gs3, os0, os1, os2, os3):
    rbuf = [r0, r1, r2, r3]
    gsem = [gs0, gs1, gs2, gs3]
    osem = [os0, os1, os2, os3]

    wid = lax.axis_index("s") * NC + lax.axis_index("c")
    col0 = wid * B_PER_W

    idx_cp = pltpu.make_async_copy(
        data_hbm.at[:, pl.ds(col0, B_PER_W)], idx_all, idx_sem)
    idx_cp.start()
    pltpu.sync_copy(pos_hbm, pos_v)
    idx_cp.wait()

    def prefill(b, o):
        pv = [pos_v[o, pl.ds(16 * j, 16)] for j in range(4)]

        def row_body(r, carry):
            for j in range(4):
                rbuf[b][r, pl.ds(16 * j, 16)] = pv[j]
            return carry

        lax.fori_loop(0, CHUNK, row_body, 0, unroll=4)

    def fire_gather(b, o):
        pltpu.async_copy(
            table_hbm.at[idx_all.at[o, pl.ds(CHUNK * b, CHUNK)]],
            rbuf[b], gsem[b], add=True)

    for b in range(NCHUNK):
        prefill(b, 0)
        fire_gather(b, 0)

    def outer(o, carry):
        for b in range(NCHUNK):
            pltpu.make_async_copy(
                table_hbm.at[idx_all.at[o, pl.ds(CHUNK * b, CHUNK)]],
                rbuf[b], gsem[b]).wait()

            out_cp = pltpu.make_async_copy(
                rbuf[b],
                out_hbm.at[o, pl.ds(col0 + CHUNK * b, CHUNK), :],
                osem[b])
            out_cp.start()

            @pl.when(o < SEQ - 1)
            def _(o=o, b=b, out_cp=out_cp):
                out_cp.wait()
                prefill(b, o + 1)
                fire_gather(b, o + 1)
        return carry

    lax.fori_loop(0, SEQ, outer, 0)

    for b in range(NCHUNK):
        pltpu.make_async_copy(
            rbuf[b], out_hbm.at[0, pl.ds(0, CHUNK), :], osem[b]).wait()


def _to_tiles_body(i_ref, o_ref):
    y = i_ref[0, :, 0:EMB]
    z = jnp.swapaxes(y, 0, 1)
    z = z.reshape(8, 8, 64, 128)
    o_ref[0] = z.transpose(0, 2, 1, 3)


_to_tiles = pl.pallas_call(
    _to_tiles_body,
    grid=(SEQ, 2),
    in_specs=[pl.BlockSpec((1, 8192, 128), lambda s, j: (s, j, 0))],
    out_specs=pl.BlockSpec((1, 8, 64, 8, 128), lambda s, j: (s, 0, j, 0, 0)),
    out_shape=jax.ShapeDtypeStruct((SEQ, 8, BATCH // 128, 8, 128),
                                   jnp.float32),
)


def kernel(data, token_table, position_table):
    eye = jnp.eye(EMB, dtype=jnp.float32) * SCALE
    scaled = _relayout(token_table.T, eye)
    ipad = _emb_kernel(data.astype(jnp.int32), scaled, position_table)
    out5d = _to_tiles(ipad)
    return out5d.transpose(0, 2, 4, 1, 3).reshape(SEQ, BATCH, EMB)


# speedup vs baseline: 1.2443x; 1.0320x over previous
import functools
import math

import jax
import jax.numpy as jnp
from jax import lax
from jax.experimental import pallas as pl
from jax.experimental.pallas import tpu as pltpu
from jax.experimental.pallas import tpu_sc as plsc

SEQ = 50
BATCH = 16384
EMB = 64
VOCAB = 1000000
SCALE = math.sqrt(EMB)

NC = 2
NS = 16
NW = NC * NS

B_PER_W = BATCH // NW
CHUNK = 128
NCHUNK = B_PER_W // CHUNK

TBLK = 16384
TGRID = -(-VOCAB // TBLK)


def _relayout_body(tt_ref, eye_ref, l_ref):
    x = tt_ref[...]
    y = lax.dot_general(x, eye_ref[...],
                        (((0,), (0,)), ((), ())),
                        preferred_element_type=jnp.float32)
    l_ref[:, 0:EMB] = y


_relayout = pl.pallas_call(
    _relayout_body,
    grid=(TGRID,),
    in_specs=[
        pl.BlockSpec((EMB, TBLK), lambda i: (0, i)),
        pl.BlockSpec((EMB, EMB), lambda i: (0, 0)),
    ],
    out_specs=pl.BlockSpec((TBLK, 128), lambda i: (i, 0)),
    out_shape=jax.ShapeDtypeStruct((VOCAB, 128), jnp.float32),
)

_mesh = plsc.VectorSubcoreMesh(core_axis_name="c", subcore_axis_name="s")


@functools.partial(
    pl.kernel,
    out_type=jax.ShapeDtypeStruct((SEQ, BATCH, 128), jnp.float32),
    mesh=_mesh,
    compiler_params=pltpu.CompilerParams(use_tc_tiling_on_sc=False),
    scratch_types=(
        [pltpu.VMEM((SEQ, B_PER_W), jnp.int32)]
        + [pltpu.VMEM((CHUNK, 128), jnp.float32) for _ in range(NCHUNK)]
        + [pltpu.VMEM((SEQ, EMB), jnp.float32)]
        + [pltpu.SemaphoreType.DMA]
        + [pltpu.SemaphoreType.DMA for _ in range(NCHUNK)]
        + [pltpu.SemaphoreType.DMA for _ in range(NCHUNK)]
    ),
)
def _emb_kernel(data_hbm, table_hbm, pos_hbm, out_hbm,
                idx_all, r0, r1, r2, r3, pos_v,
                idx_sem, gs0, gs1, gs2, gs3, os0, os1, os2, os3):
    rbuf = [r0, r1, r2, r3]
    gsem = [gs0, gs1, gs2, gs3]
    osem = [os0, os1, os2, os3]

    wid = lax.axis_index("s") * NC + lax.axis_index("c")
    col0 = wid * B_PER_W

    idx_cp = pltpu.make_async_copy(
        data_hbm.at[:, pl.ds(col0, B_PER_W)], idx_all, idx_sem)
    idx_cp.start()
    pltpu.sync_copy(pos_hbm, pos_v)
    idx_cp.wait()

    def prefill(b, o):
        pv = [pos_v[o, pl.ds(16 * j, 16)] for j in range(4)]

        def row_body(r, carry):
            for j in range(4):
                rbuf[b][r, pl.ds(16 * j, 16)] = pv[j]
            return carry

        lax.fori_loop(0, CHUNK, row_body, 0, unroll=4)

    def fire_gather(b, o):
        pltpu.async_copy(
            table_hbm.at[idx_all.at[o, pl.ds(CHUNK * b, CHUNK)]],
            rbuf[b], gsem[b], add=True)

    for b in range(NCHUNK):
        prefill(b, 0)
        fire_gather(b, 0)

    def outer(o, carry):
        for b in range(NCHUNK):
            pltpu.make_async_copy(
                table_hbm.at[idx_all.at[o, pl.ds(CHUNK * b, CHUNK)]],
                rbuf[b], gsem[b]).wait()

            out_cp = pltpu.make_async_copy(
                rbuf[b],
                out_hbm.at[o, pl.ds(col0 + CHUNK * b, CHUNK), :],
                osem[b])
            out_cp.start()

            @pl.when(o < SEQ - 1)
            def _(o=o, b=b, out_cp=out_cp):
                out_cp.wait()
                prefill(b, o + 1)
                fire_gather(b, o + 1)
        return carry

    lax.fori_loop(0, SEQ, outer, 0)

    for b in range(NCHUNK):
        pltpu.make_async_copy(
            rbuf[b], out_hbm.at[0, pl.ds(0, CHUNK), :], osem[b]).wait()


def _to_tiles_body(i_ref, o_ref):
    y = i_ref[0, :, 0:EMB]
    z = jnp.swapaxes(y, 0, 1)
    z = z.reshape(8, 8, 64, 128)
    o_ref[0] = z.transpose(0, 2, 1, 3)


_to_tiles = pl.pallas_call(
    _to_tiles_body,
    grid=(SEQ, 2),
    in_specs=[pl.BlockSpec((1, 8192, 128), lambda s, j: (s, j, 0))],
    out_specs=pl.BlockSpec((1, 8, 64, 8, 128), lambda s, j: (s, 0, j, 0, 0)),
    out_shape=jax.ShapeDtypeStruct((SEQ, 8, BATCH // 128, 8, 128),
                                   jnp.float32),
)


def kernel(data, token_table, position_table):
    eye = jnp.eye(EMB, dtype=jnp.float32) * SCALE
    scaled = _relayout(token_table.T, eye)
    ipad = _emb_kernel(data.astype(jnp.int32), scaled, position_table)
    out5d = _to_tiles(ipad)
    return out5d.transpose(0, 2, 4, 1, 3).reshape(SEQ, BATCH, EMB)
